# baseline (speedup 1.0000x reference)
import functools

import jax
import jax.numpy as jnp
from jax import lax
from jax.experimental import pallas as pl
from jax.experimental.pallas import tpu as pltpu
from jax.experimental.pallas import tpu_sc as plsc

N_NODES = 10000
N_EDGES = 320000
D = 128
HALF1 = 166400
HALFS = (HALF1, N_EDGES - HALF1)
HBASE = (0, HALF1)

NC = 2
NS = 16
NPAD = 10240
STRIPE = NPAD // NS

EPT_F = N_EDGES // (NC * NS)
BLK = 80
NB_F = EPT_F // BLK
NSLOT_S = 3
NGRP_S = 40

EPT_H = tuple(n // (NC * NS) for n in HALFS)
NB_H = tuple(e // BLK for e in EPT_H)
NSLOT = 5

_mesh = plsc.VectorSubcoreMesh(core_axis_name="c", subcore_axis_name="s")


def _half_idx(a, h):
    return a[HBASE[h]:HBASE[h] + HALFS[h]].reshape(NC, NS, NB_H[h], BLK)


def _full_idx(a):
    return a.reshape(NC, NS, NB_F, BLK)


def _make_init(h):
    ept, nb = EPT_H[h], NB_H[h]
    ngrp = nb // NSLOT

    @functools.partial(
        pl.kernel,
        out_type=jax.ShapeDtypeStruct((HALFS[h], D), jnp.float32),
        mesh=_mesh,
        scratch_types=[
            pltpu.VMEM((nb, BLK), jnp.int32),
            pltpu.VMEM((NSLOT, BLK, D), jnp.float32),
            pltpu.SemaphoreType.DMA((NSLOT,)),
            pltpu.SemaphoreType.DMA((NSLOT,)),
            pltpu.SemaphoreType.DMA((NSLOT,)),
        ],
    )
    def init_k(e_hbm, v_hbm, srcp, h0, idx_v, bufs, sa, sb, sc):
        c = lax.axis_index("c")
        s = lax.axis_index("s")
        base = (c * NS + s) * ept
        ebase = HBASE[h] + base
        pltpu.sync_copy(srcp.at[c, s], idx_v)

        def a_issue(j, p):
            pltpu.async_copy(e_hbm.at[pl.ds(ebase + j * BLK, BLK), :],
                             bufs.at[p], sa.at[p])

        def a_wait(p):
            pltpu.make_async_copy(e_hbm.at[pl.ds(ebase, BLK), :],
                                  bufs.at[p], sa.at[p]).wait()

        def b_issue(j, p):
            pltpu.async_copy(v_hbm.at[idx_v.at[j]], bufs.at[p], sb.at[p],
                             add=True)

        def b_wait(p):
            pltpu.make_async_copy(v_hbm.at[idx_v.at[0]], bufs.at[p],
                                  sb.at[p]).wait()

        def c_issue(j, p):
            pltpu.async_copy(bufs.at[p],
                             h0.at[pl.ds(base + j * BLK, BLK), :],
                             sc.at[p])

        def c_wait(p):
            pltpu.make_async_copy(bufs.at[p], h0.at[pl.ds(base, BLK), :],
                                  sc.at[p]).wait()

        for p in range(NSLOT):
            a_issue(p, p)

        def body(k, _):
            for p in range(NSLOT):
                j = k * NSLOT + p
                a_wait(p)
                b_issue(j, p)
            for p in range(NSLOT):
                j = k * NSLOT + p
                b_wait(p)
                c_issue(j, p)
                c_wait(p)
                a_issue(j + NSLOT, p)
            return _

        lax.fori_loop(0, ngrp - 1, body, None)
        for p in range(NSLOT):
            a_wait(p)
            b_issue((ngrp - 1) * NSLOT + p, p)
        for p in range(NSLOT):
            b_wait(p)
            c_issue((ngrp - 1) * NSLOT + p, p)
            c_wait(p)

    return init_k


def _make_gather(h):
    ept, nb = EPT_H[h], NB_H[h]
    ngrp = nb // NSLOT

    @functools.partial(
        pl.kernel,
        out_type=jax.ShapeDtypeStruct((HALFS[h], D), jnp.float32),
        mesh=_mesh,
        scratch_types=[
            pltpu.VMEM((nb, BLK), jnp.int32),
            pltpu.VMEM((nb, BLK), jnp.int32),
            pltpu.VMEM((NSLOT, BLK, D), jnp.float32),
            pltpu.SemaphoreType.DMA((NSLOT,)),
            pltpu.SemaphoreType.DMA((NSLOT,)),
            pltpu.SemaphoreType.DMA((NSLOT,)),
        ],
    )
    def gather_k(mv, hneg, srcp, revp, out, src_v, rev_v, bufs, sa, sb, sc):
        c = lax.axis_index("c")
        s = lax.axis_index("s")
        base = (c * NS + s) * ept
        pltpu.sync_copy(srcp.at[c, s], src_v)
        pltpu.sync_copy(revp.at[c, s], rev_v)

        def a_issue(j, p):
            pltpu.async_copy(mv.at[src_v.at[j]], bufs.at[p], sa.at[p])

        def a_wait(p):
            pltpu.make_async_copy(mv.at[src_v.at[0]], bufs.at[p],
                                  sa.at[p]).wait()

        def b_issue(j, p):
            pltpu.async_copy(hneg.at[rev_v.at[j]], bufs.at[p], sb.at[p],
                             add=True)

        def b_wait(p):
            pltpu.make_async_copy(hneg.at[rev_v.at[0]], bufs.at[p],
                                  sb.at[p]).wait()

        def c_issue(j, p):
            pltpu.async_copy(bufs.at[p],
                             out.at[pl.ds(base + j * BLK, BLK), :],
                             sc.at[p])

        def c_wait(p):
            pltpu.make_async_copy(bufs.at[p], out.at[pl.ds(base, BLK), :],
                                  sc.at[p]).wait()

        for p in range(NSLOT):
            a_issue(p, p)

        def body(k, _):
            for p in range(NSLOT):
                j = k * NSLOT + p
                a_wait(p)
                b_issue(j, p)
            for p in range(NSLOT):
                j = k * NSLOT + p
                b_wait(p)
                c_issue(j, p)
                c_wait(p)
                a_issue(j + NSLOT, p)
            return _

        lax.fori_loop(0, ngrp - 1, body, None)
        for p in range(NSLOT):
            a_wait(p)
            b_issue((ngrp - 1) * NSLOT + p, p)
        for p in range(NSLOT):
            b_wait(p)
            c_issue((ngrp - 1) * NSLOT + p, p)
            c_wait(p)

    return gather_k


_sc_init_h = [_make_init(0), _make_init(1)]
_sc_gather_h = [_make_gather(0), _make_gather(1)]


@functools.partial(
    pl.kernel,
    out_type=jax.ShapeDtypeStruct((NC, NPAD, D), jnp.float32),
    mesh=_mesh,
    scratch_types=[
        pltpu.VMEM((NB_F, BLK), jnp.int32),
        pltpu.VMEM((NSLOT_S, BLK, D), jnp.float32),
        pltpu.VMEM_SHARED((NPAD, D), jnp.float32),
        pltpu.SemaphoreType.DMA((NSLOT_S,)),
        pltpu.SemaphoreType.DMA((NSLOT_S,)),
    ],
)
def _sc_scatter(data, idxp, zeros, out, idx_v, bufs, acc_sh, sa, sb):
    c = lax.axis_index("c")
    s = lax.axis_index("s")
    base = (c * NS + s) * EPT_F
    pltpu.sync_copy(zeros.at[pl.ds(s * STRIPE, STRIPE)],
                    acc_sh.at[pl.ds(s * STRIPE, STRIPE)])
    pltpu.sync_copy(idxp.at[c, s], idx_v)
    plsc.subcore_barrier()

    def a_issue(j, p):
        pltpu.async_copy(data.at[pl.ds(base + j * BLK, BLK), :],
                         bufs.at[p], sa.at[p])

    def a_wait(p):
        pltpu.make_async_copy(data.at[pl.ds(base, BLK), :],
                              bufs.at[p], sa.at[p]).wait()

    def b_issue(j, p):
        pltpu.async_copy(bufs.at[p], acc_sh.at[idx_v.at[j]], sb.at[p],
                         add=True)

    def b_wait(p):
        pltpu.make_async_copy(bufs.at[p], acc_sh.at[idx_v.at[0]],
                              sb.at[p]).wait()

    for p in range(NSLOT_S):
        a_issue(p, p)

    def body(k, _):
        for p in range(NSLOT_S):
            j = k * NSLOT_S + p
            a_wait(p)
            b_issue(j, p)
        for p in range(NSLOT_S):
            b_wait(p)
            a_issue(k * NSLOT_S + p + NSLOT_S, p)
        return _

    lax.fori_loop(0, NGRP_S, body, None)
    e0 = NGRP_S * NSLOT_S
    for p in range(NSLOT_S):
        a_wait(p)
        b_issue(e0 + p, p)
    for p in range(NB_F - e0 - NSLOT_S):
        b_wait(p)
        a_issue(e0 + NSLOT_S + p, p)
    b_wait(NSLOT_S - 1)
    for p in range(NB_F - e0 - NSLOT_S):
        a_wait(p)
        b_issue(e0 + NSLOT_S + p, p)
    for p in range(NB_F - e0 - NSLOT_S):
        b_wait(p)
    plsc.subcore_barrier()
    pltpu.sync_copy(acc_sh.at[pl.ds(s * STRIPE, STRIPE)],
                    out.at[c, pl.ds(s * STRIPE, STRIPE)])


BE = 1600
HGRIDS = tuple((n // BE,) for n in HALFS)
OFF1 = HALF1 // BE


def _tc_combine_body(p_ref, o_ref):
    o_ref[...] = -(p_ref[0] + p_ref[1])


def _tc_out_body(p_ref, o_ref):
    o_ref[...] = p_ref[0] + p_ref[1]


_half_spec = pl.BlockSpec((BE, D), lambda i: (i, 0))
_w_spec = pl.BlockSpec((D, D), lambda i: (0, 0))
_b_spec = pl.BlockSpec((1, D), lambda i: (0, 0))
_alias_spec = pl.BlockSpec((8, D), lambda i: (0, 0))


def _make_relu_neg(h):
    if h == 0:
        def body0(x_ref, o_ref):
            o_ref[...] = -jnp.maximum(x_ref[...], 0.0)

        return pl.pallas_call(
            body0,
            grid=HGRIDS[0],
            in_specs=[_half_spec],
            out_specs=pl.BlockSpec((BE, D), lambda i: (i, 0)),
            out_shape=jax.ShapeDtypeStruct((N_EDGES, D), jnp.float32),
        )

    def body1(x_ref, prev_ref, o_ref):
        del prev_ref
        o_ref[...] = -jnp.maximum(x_ref[...], 0.0)

    return pl.pallas_call(
        body1,
        grid=HGRIDS[1],
        in_specs=[_half_spec, _alias_spec],
        out_specs=pl.BlockSpec((BE, D), lambda i: (i + OFF1, 0)),
        out_shape=jax.ShapeDtypeStruct((N_EDGES, D), jnp.float32),
        input_output_aliases={1: 0},
    )


def _make_layer(h):
    def compute(m_ref, h_ref, w_ref, b_ref):
        hn = (h_ref[...] + b_ref[...]
              + jnp.dot(m_ref[...], w_ref[...],
                        preferred_element_type=jnp.float32))
        return hn

    if h == 0:
        def body0(m_ref, h_ref, w_ref, b_ref, hn_ref, hneg_ref):
            hn = compute(m_ref, h_ref, w_ref, b_ref)
            hn_ref[...] = hn
            hneg_ref[...] = -jnp.maximum(hn, 0.0)

        return pl.pallas_call(
            body0,
            grid=HGRIDS[0],
            in_specs=[_half_spec, _half_spec, _w_spec, _b_spec],
            out_specs=(_half_spec, pl.BlockSpec((BE, D), lambda i: (i, 0))),
            out_shape=(jax.ShapeDtypeStruct((HALFS[0], D), jnp.float32),
                       jax.ShapeDtypeStruct((N_EDGES, D), jnp.float32)),
        )

    def body1(m_ref, h_ref, w_ref, b_ref, prev_ref, hn_ref, hneg_ref):
        del prev_ref
        hn = compute(m_ref, h_ref, w_ref, b_ref)
        hn_ref[...] = hn
        hneg_ref[...] = -jnp.maximum(hn, 0.0)

    return pl.pallas_call(
        body1,
        grid=HGRIDS[1],
        in_specs=[_half_spec, _half_spec, _w_spec, _b_spec, _alias_spec],
        out_specs=(_half_spec,
                   pl.BlockSpec((BE, D), lambda i: (i + OFF1, 0))),
        out_shape=(jax.ShapeDtypeStruct((HALFS[1], D), jnp.float32),
                   jax.ShapeDtypeStruct((N_EDGES, D), jnp.float32)),
        input_output_aliases={4: 1},
    )


def _make_layer_last(h):
    if h == 0:
        def body0(m_ref, h_ref, w_ref, b_ref, hn_ref):
            hn_ref[...] = (h_ref[...] + b_ref[...]
                           + jnp.dot(m_ref[...], w_ref[...],
                                     preferred_element_type=jnp.float32))

        return pl.pallas_call(
            body0,
            grid=HGRIDS[0],
            in_specs=[_half_spec, _half_spec, _w_spec, _b_spec],
            out_specs=pl.BlockSpec((BE, D), lambda i: (i, 0)),
            out_shape=jax.ShapeDtypeStruct((N_EDGES, D), jnp.float32),
        )

    def body1(m_ref, h_ref, w_ref, b_ref, prev_ref, hn_ref):
        del prev_ref
        hn_ref[...] = (h_ref[...] + b_ref[...]
                       + jnp.dot(m_ref[...], w_ref[...],
                                 preferred_element_type=jnp.float32))

    return pl.pallas_call(
        body1,
        grid=HGRIDS[1],
        in_specs=[_half_spec, _half_spec, _w_spec, _b_spec, _alias_spec],
        out_specs=pl.BlockSpec((BE, D), lambda i: (i + OFF1, 0)),
        out_shape=jax.ShapeDtypeStruct((N_EDGES, D), jnp.float32),
        input_output_aliases={4: 0},
    )


_tc_relu_neg_h = [_make_relu_neg(0), _make_relu_neg(1)]
_tc_layer_h = [_make_layer(0), _make_layer(1)]
_tc_layer_last_h = [_make_layer_last(0), _make_layer_last(1)]

_tc_combine = pl.pallas_call(
    _tc_combine_body,
    grid=(8,),
    in_specs=[pl.BlockSpec((NC, NPAD // 8, D), lambda i: (0, i, 0))],
    out_specs=pl.BlockSpec((NPAD // 8, D), lambda i: (i, 0)),
    out_shape=jax.ShapeDtypeStruct((NPAD, D), jnp.float32),
)

_tc_out = pl.pallas_call(
    _tc_out_body,
    grid=(10,),
    in_specs=[pl.BlockSpec((NC, N_NODES // 10, D), lambda i: (0, i, 0))],
    out_specs=pl.BlockSpec((N_NODES // 10, D), lambda i: (i, 0)),
    out_shape=jax.ShapeDtypeStruct((N_NODES, D), jnp.float32),
)


def kernel(V, E, edge_index, rev_index, W1, b1, W2, b2, W3, b3):
    src = edge_index[0]
    dest = edge_index[1]
    srcp_h = [_half_idx(src, h) for h in range(2)]
    revp_h = [_half_idx(rev_index, h) for h in range(2)]
    destp = _full_idx(dest)
    srcp_f = _full_idx(src)
    zeros = jnp.zeros((NPAD, D), jnp.float32)

    h_half = [_sc_init_h[h](E, V, srcp_h[h]) for h in range(2)]
    hneg_v = _tc_relu_neg_h[0](h_half[0])
    hneg = _tc_relu_neg_h[1](h_half[1], hneg_v)

    params = [(W1, b1.reshape(1, D)), (W2, b2.reshape(1, D)),
              (W3, b3.reshape(1, D))]
    for li, (w, b2d) in enumerate(params):
        parts = _sc_scatter(hneg, destp, zeros)
        mv = _tc_combine(parts)
        m_half = [_sc_gather_h[h](mv, hneg, srcp_h[h], revp_h[h])
                  for h in range(2)]
        if li < 2:
            h0n, hneg_v = _tc_layer_h[0](m_half[0], h_half[0], w, b2d)
            h1n, hneg = _tc_layer_h[1](m_half[1], h_half[1], w, b2d, hneg_v)
            h_half = [h0n, h1n]
        else:
            h_v = _tc_layer_last_h[0](m_half[0], h_half[0], w, b2d)
            h_full = _tc_layer_last_h[1](m_half[1], h_half[1], w, b2d, h_v)
    parts = _sc_scatter(h_full, srcp_f, zeros)
    v_out = _tc_out(parts)
    return (v_out, h_full)


# speedup vs baseline: 1.0490x; 1.0490x over previous
import functools

import jax
import jax.numpy as jnp
from jax import lax
from jax.experimental import pallas as pl
from jax.experimental.pallas import tpu as pltpu
from jax.experimental.pallas import tpu_sc as plsc

N_NODES = 10000
N_EDGES = 320000
D = 128
HALF1 = 166400
HALFS = (HALF1, N_EDGES - HALF1)
HBASE = (0, HALF1)

NC = 2
NS = 16
NPAD = 10240
STRIPE = NPAD // NS

EPT_F = N_EDGES // (NC * NS)
BLK = 80
NB_F = EPT_F // BLK
NSLOT_S = 3
NGRP_S = 40

EPT_H = tuple(n // (NC * NS) for n in HALFS)
NB_H = tuple(e // BLK for e in EPT_H)
NSLOT = 5

_mesh = plsc.VectorSubcoreMesh(core_axis_name="c", subcore_axis_name="s")


def _half_idx(a, h):
    return a[HBASE[h]:HBASE[h] + HALFS[h]].reshape(NC, NS, NB_H[h], BLK)


def _full_idx(a):
    return a.reshape(NC, NS, NB_F, BLK)


def _make_init(h):
    ept, nb = EPT_H[h], NB_H[h]
    ngrp = nb // NSLOT

    @functools.partial(
        pl.kernel,
        out_type=jax.ShapeDtypeStruct((HALFS[h], D), jnp.float32),
        mesh=_mesh,
        scratch_types=[
            pltpu.VMEM((nb, BLK), jnp.int32),
            pltpu.VMEM((NSLOT, BLK, D), jnp.float32),
            pltpu.SemaphoreType.DMA((NSLOT,)),
            pltpu.SemaphoreType.DMA((NSLOT,)),
            pltpu.SemaphoreType.DMA((NSLOT,)),
        ],
    )
    def init_k(e_hbm, v_hbm, srcp, h0, idx_v, bufs, sa, sb, sc):
        c = lax.axis_index("c")
        s = lax.axis_index("s")
        base = (c * NS + s) * ept
        ebase = HBASE[h] + base
        pltpu.sync_copy(srcp.at[c, s], idx_v)

        def a_issue(j, p):
            pltpu.async_copy(e_hbm.at[pl.ds(ebase + j * BLK, BLK), :],
                             bufs.at[p], sa.at[p])

        def a_wait(p):
            pltpu.make_async_copy(e_hbm.at[pl.ds(ebase, BLK), :],
                                  bufs.at[p], sa.at[p]).wait()

        def b_issue(j, p):
            pltpu.async_copy(v_hbm.at[idx_v.at[j]], bufs.at[p], sb.at[p],
                             add=True)

        def b_wait(p):
            pltpu.make_async_copy(v_hbm.at[idx_v.at[0]], bufs.at[p],
                                  sb.at[p]).wait()

        def c_issue(j, p):
            pltpu.async_copy(bufs.at[p],
                             h0.at[pl.ds(base + j * BLK, BLK), :],
                             sc.at[p])

        def c_wait(p):
            pltpu.make_async_copy(bufs.at[p], h0.at[pl.ds(base, BLK), :],
                                  sc.at[p]).wait()

        for p in range(NSLOT):
            a_issue(p, p)

        def body(k, _):
            for p in range(NSLOT):
                j = k * NSLOT + p
                a_wait(p)
                b_issue(j, p)
            for p in range(NSLOT):
                j = k * NSLOT + p
                b_wait(p)
                c_issue(j, p)
                c_wait(p)
                a_issue(j + NSLOT, p)
            return _

        lax.fori_loop(0, ngrp - 1, body, None)
        for p in range(NSLOT):
            a_wait(p)
            b_issue((ngrp - 1) * NSLOT + p, p)
        for p in range(NSLOT):
            b_wait(p)
            c_issue((ngrp - 1) * NSLOT + p, p)
            c_wait(p)

    return init_k


NSLOT_G = 3
MVR = N_NODES // NS


def _make_gather(h):
    ept, nb = EPT_H[h], NB_H[h]
    ngrp = nb // NSLOT_G
    rem = nb - ngrp * NSLOT_G

    @functools.partial(
        pl.kernel,
        out_type=jax.ShapeDtypeStruct((HALFS[h], D), jnp.float32),
        mesh=_mesh,
        scratch_types=[
            pltpu.VMEM((nb, BLK), jnp.int32),
            pltpu.VMEM((nb, BLK), jnp.int32),
            pltpu.VMEM((NSLOT_G, BLK, D), jnp.float32),
            pltpu.VMEM_SHARED((N_NODES, D), jnp.float32),
            pltpu.SemaphoreType.DMA((NSLOT_G,)),
            pltpu.SemaphoreType.DMA((NSLOT_G,)),
            pltpu.SemaphoreType.DMA((NSLOT_G,)),
        ],
    )
    def gather_k(mv, hneg, srcp, revp, out, src_v, rev_v, bufs, mv_sh,
                 sa, sb, sc):
        c = lax.axis_index("c")
        s = lax.axis_index("s")
        base = (c * NS + s) * ept
        pltpu.sync_copy(mv.at[pl.ds(s * 624, 624)],
                        mv_sh.at[pl.ds(s * 624, 624)])

        @pl.when(s == NS - 1)
        def _():
            pltpu.sync_copy(mv.at[pl.ds(9984, 16)],
                            mv_sh.at[pl.ds(9984, 16)])

        pltpu.sync_copy(srcp.at[c, s], src_v)
        pltpu.sync_copy(revp.at[c, s], rev_v)
        plsc.subcore_barrier()

        def a_issue(j, p):
            pltpu.async_copy(mv_sh.at[src_v.at[j]], bufs.at[p], sa.at[p])

        def a_wait(p):
            pltpu.make_async_copy(mv_sh.at[src_v.at[0]], bufs.at[p],
                                  sa.at[p]).wait()

        def b_issue(j, p):
            pltpu.async_copy(hneg.at[rev_v.at[j]], bufs.at[p], sb.at[p],
                             add=True)

        def b_wait(p):
            pltpu.make_async_copy(hneg.at[rev_v.at[0]], bufs.at[p],
                                  sb.at[p]).wait()

        def c_issue(j, p):
            pltpu.async_copy(bufs.at[p],
                             out.at[pl.ds(base + j * BLK, BLK), :],
                             sc.at[p])

        def c_wait(p):
            pltpu.make_async_copy(bufs.at[p], out.at[pl.ds(base, BLK), :],
                                  sc.at[p]).wait()

        for p in range(NSLOT_G):
            a_issue(p, p)

        def body(k, _):
            for p in range(NSLOT_G):
                j = k * NSLOT_G + p
                a_wait(p)
                b_issue(j, p)
            for p in range(NSLOT_G):
                j = k * NSLOT_G + p
                b_wait(p)
                c_issue(j, p)
                c_wait(p)
                a_issue(j + NSLOT_G, p)
            return _

        lax.fori_loop(0, ngrp - 1, body, None)
        e0 = (ngrp - 1) * NSLOT_G
        for p in range(NSLOT_G):
            a_wait(p)
            b_issue(e0 + p, p)
        for p in range(NSLOT_G):
            b_wait(p)
            c_issue(e0 + p, p)
            c_wait(p)
            if p < rem:
                a_issue(e0 + NSLOT_G + p, p)
        for p in range(rem):
            a_wait(p)
            b_issue(e0 + NSLOT_G + p, p)
        for p in range(rem):
            b_wait(p)
            c_issue(e0 + NSLOT_G + p, p)
            c_wait(p)

    return gather_k


_sc_init_h = [_make_init(0), _make_init(1)]
_sc_gather_h = [_make_gather(0), _make_gather(1)]


@functools.partial(
    pl.kernel,
    out_type=jax.ShapeDtypeStruct((NC, NPAD, D), jnp.float32),
    mesh=_mesh,
    scratch_types=[
        pltpu.VMEM((NB_F, BLK), jnp.int32),
        pltpu.VMEM((NSLOT_S, BLK, D), jnp.float32),
        pltpu.VMEM_SHARED((NPAD, D), jnp.float32),
        pltpu.SemaphoreType.DMA((NSLOT_S,)),
        pltpu.SemaphoreType.DMA((NSLOT_S,)),
    ],
)
def _sc_scatter(data, idxp, zeros, out, idx_v, bufs, acc_sh, sa, sb):
    c = lax.axis_index("c")
    s = lax.axis_index("s")
    base = (c * NS + s) * EPT_F
    pltpu.sync_copy(zeros.at[pl.ds(s * STRIPE, STRIPE)],
                    acc_sh.at[pl.ds(s * STRIPE, STRIPE)])
    pltpu.sync_copy(idxp.at[c, s], idx_v)
    plsc.subcore_barrier()

    def a_issue(j, p):
        pltpu.async_copy(data.at[pl.ds(base + j * BLK, BLK), :],
                         bufs.at[p], sa.at[p])

    def a_wait(p):
        pltpu.make_async_copy(data.at[pl.ds(base, BLK), :],
                              bufs.at[p], sa.at[p]).wait()

    def b_issue(j, p):
        pltpu.async_copy(bufs.at[p], acc_sh.at[idx_v.at[j]], sb.at[p],
                         add=True)

    def b_wait(p):
        pltpu.make_async_copy(bufs.at[p], acc_sh.at[idx_v.at[0]],
                              sb.at[p]).wait()

    for p in range(NSLOT_S):
        a_issue(p, p)

    def body(k, _):
        for p in range(NSLOT_S):
            j = k * NSLOT_S + p
            a_wait(p)
            b_issue(j, p)
        for p in range(NSLOT_S):
            b_wait(p)
            a_issue(k * NSLOT_S + p + NSLOT_S, p)
        return _

    lax.fori_loop(0, NGRP_S, body, None)
    e0 = NGRP_S * NSLOT_S
    for p in range(NSLOT_S):
        a_wait(p)
        b_issue(e0 + p, p)
    for p in range(NB_F - e0 - NSLOT_S):
        b_wait(p)
        a_issue(e0 + NSLOT_S + p, p)
    b_wait(NSLOT_S - 1)
    for p in range(NB_F - e0 - NSLOT_S):
        a_wait(p)
        b_issue(e0 + NSLOT_S + p, p)
    for p in range(NB_F - e0 - NSLOT_S):
        b_wait(p)
    plsc.subcore_barrier()
    pltpu.sync_copy(acc_sh.at[pl.ds(s * STRIPE, STRIPE)],
                    out.at[c, pl.ds(s * STRIPE, STRIPE)])


BE = 1600
HGRIDS = tuple((n // BE,) for n in HALFS)
OFF1 = HALF1 // BE


def _tc_combine_body(p_ref, o_ref):
    o_ref[...] = -(p_ref[0] + p_ref[1])


def _tc_out_body(p_ref, o_ref):
    o_ref[...] = p_ref[0] + p_ref[1]


_half_spec = pl.BlockSpec((BE, D), lambda i: (i, 0))
_w_spec = pl.BlockSpec((D, D), lambda i: (0, 0))
_b_spec = pl.BlockSpec((1, D), lambda i: (0, 0))
_alias_spec = pl.BlockSpec((8, D), lambda i: (0, 0))


def _make_relu_neg(h):
    if h == 0:
        def body0(x_ref, o_ref):
            o_ref[...] = -jnp.maximum(x_ref[...], 0.0)

        return pl.pallas_call(
            body0,
            grid=HGRIDS[0],
            in_specs=[_half_spec],
            out_specs=pl.BlockSpec((BE, D), lambda i: (i, 0)),
            out_shape=jax.ShapeDtypeStruct((N_EDGES, D), jnp.float32),
        )

    def body1(x_ref, prev_ref, o_ref):
        del prev_ref
        o_ref[...] = -jnp.maximum(x_ref[...], 0.0)

    return pl.pallas_call(
        body1,
        grid=HGRIDS[1],
        in_specs=[_half_spec, _alias_spec],
        out_specs=pl.BlockSpec((BE, D), lambda i: (i + OFF1, 0)),
        out_shape=jax.ShapeDtypeStruct((N_EDGES, D), jnp.float32),
        input_output_aliases={1: 0},
    )


def _make_layer(h):
    def compute(m_ref, h_ref, w_ref, b_ref):
        hn = (h_ref[...] + b_ref[...]
              + jnp.dot(m_ref[...], w_ref[...],
                        preferred_element_type=jnp.float32))
        return hn

    if h == 0:
        def body0(m_ref, h_ref, w_ref, b_ref, hn_ref, hneg_ref):
            hn = compute(m_ref, h_ref, w_ref, b_ref)
            hn_ref[...] = hn
            hneg_ref[...] = -jnp.maximum(hn, 0.0)

        return pl.pallas_call(
            body0,
            grid=HGRIDS[0],
            in_specs=[_half_spec, _half_spec, _w_spec, _b_spec],
            out_specs=(_half_spec, pl.BlockSpec((BE, D), lambda i: (i, 0))),
            out_shape=(jax.ShapeDtypeStruct((HALFS[0], D), jnp.float32),
                       jax.ShapeDtypeStruct((N_EDGES, D), jnp.float32)),
        )

    def body1(m_ref, h_ref, w_ref, b_ref, prev_ref, hn_ref, hneg_ref):
        del prev_ref
        hn = compute(m_ref, h_ref, w_ref, b_ref)
        hn_ref[...] = hn
        hneg_ref[...] = -jnp.maximum(hn, 0.0)

    return pl.pallas_call(
        body1,
        grid=HGRIDS[1],
        in_specs=[_half_spec, _half_spec, _w_spec, _b_spec, _alias_spec],
        out_specs=(_half_spec,
                   pl.BlockSpec((BE, D), lambda i: (i + OFF1, 0))),
        out_shape=(jax.ShapeDtypeStruct((HALFS[1], D), jnp.float32),
                   jax.ShapeDtypeStruct((N_EDGES, D), jnp.float32)),
        input_output_aliases={4: 1},
    )


def _make_layer_last(h):
    if h == 0:
        def body0(m_ref, h_ref, w_ref, b_ref, hn_ref):
            hn_ref[...] = (h_ref[...] + b_ref[...]
                           + jnp.dot(m_ref[...], w_ref[...],
                                     preferred_element_type=jnp.float32))

        return pl.pallas_call(
            body0,
            grid=HGRIDS[0],
            in_specs=[_half_spec, _half_spec, _w_spec, _b_spec],
            out_specs=pl.BlockSpec((BE, D), lambda i: (i, 0)),
            out_shape=jax.ShapeDtypeStruct((N_EDGES, D), jnp.float32),
        )

    def body1(m_ref, h_ref, w_ref, b_ref, prev_ref, hn_ref):
        del prev_ref
        hn_ref[...] = (h_ref[...] + b_ref[...]
                       + jnp.dot(m_ref[...], w_ref[...],
                                 preferred_element_type=jnp.float32))

    return pl.pallas_call(
        body1,
        grid=HGRIDS[1],
        in_specs=[_half_spec, _half_spec, _w_spec, _b_spec, _alias_spec],
        out_specs=pl.BlockSpec((BE, D), lambda i: (i + OFF1, 0)),
        out_shape=jax.ShapeDtypeStruct((N_EDGES, D), jnp.float32),
        input_output_aliases={4: 0},
    )


_tc_relu_neg_h = [_make_relu_neg(0), _make_relu_neg(1)]
_tc_layer_h = [_make_layer(0), _make_layer(1)]
_tc_layer_last_h = [_make_layer_last(0), _make_layer_last(1)]

_tc_combine = pl.pallas_call(
    _tc_combine_body,
    grid=(8,),
    in_specs=[pl.BlockSpec((NC, NPAD // 8, D), lambda i: (0, i, 0))],
    out_specs=pl.BlockSpec((NPAD // 8, D), lambda i: (i, 0)),
    out_shape=jax.ShapeDtypeStruct((NPAD, D), jnp.float32),
)

_tc_out = pl.pallas_call(
    _tc_out_body,
    grid=(10,),
    in_specs=[pl.BlockSpec((NC, N_NODES // 10, D), lambda i: (0, i, 0))],
    out_specs=pl.BlockSpec((N_NODES // 10, D), lambda i: (i, 0)),
    out_shape=jax.ShapeDtypeStruct((N_NODES, D), jnp.float32),
)


def kernel(V, E, edge_index, rev_index, W1, b1, W2, b2, W3, b3):
    src = edge_index[0]
    dest = edge_index[1]
    srcp_h = [_half_idx(src, h) for h in range(2)]
    revp_h = [_half_idx(rev_index, h) for h in range(2)]
    destp = _full_idx(dest)
    srcp_f = _full_idx(src)
    zeros = jnp.zeros((NPAD, D), jnp.float32)

    h_half = [_sc_init_h[h](E, V, srcp_h[h]) for h in range(2)]
    hneg_v = _tc_relu_neg_h[0](h_half[0])
    hneg = _tc_relu_neg_h[1](h_half[1], hneg_v)

    params = [(W1, b1.reshape(1, D)), (W2, b2.reshape(1, D)),
              (W3, b3.reshape(1, D))]
    for li, (w, b2d) in enumerate(params):
        parts = _sc_scatter(hneg, destp, zeros)
        mv = _tc_combine(parts)
        m_half = [_sc_gather_h[h](mv, hneg, srcp_h[h], revp_h[h])
                  for h in range(2)]
        if li < 2:
            h0n, hneg_v = _tc_layer_h[0](m_half[0], h_half[0], w, b2d)
            h1n, hneg = _tc_layer_h[1](m_half[1], h_half[1], w, b2d, hneg_v)
            h_half = [h0n, h1n]
        else:
            h_v = _tc_layer_last_h[0](m_half[0], h_half[0], w, b2d)
            h_full = _tc_layer_last_h[1](m_half[1], h_half[1], w, b2d, h_v)
    parts = _sc_scatter(h_full, srcp_f, zeros)
    v_out = _tc_out(parts)
    return (v_out, h_full)


# speedup vs baseline: 1.0607x; 1.0112x over previous
import functools

import jax
import jax.numpy as jnp
from jax import lax
from jax.experimental import pallas as pl
from jax.experimental.pallas import tpu as pltpu
from jax.experimental.pallas import tpu_sc as plsc

N_NODES = 10000
N_EDGES = 320000
D = 128
HALF1 = 166400
HALFS = (HALF1, N_EDGES - HALF1)
HBASE = (0, HALF1)

NC = 2
NS = 16
NPAD = 10240
STRIPE = NPAD // NS

EPT_F = N_EDGES // (NC * NS)
BLK = 80
NB_F = EPT_F // BLK
NSLOT_S = 3
NGRP_S = 40

EPT_H = tuple(n // (NC * NS) for n in HALFS)
NB_H = tuple(e // BLK for e in EPT_H)
NSLOT = 5

_mesh = plsc.VectorSubcoreMesh(core_axis_name="c", subcore_axis_name="s")


def _half_idx(a, h):
    return a[HBASE[h]:HBASE[h] + HALFS[h]].reshape(NC, NS, NB_H[h], BLK)


def _full_idx(a):
    return a.reshape(NC, NS, NB_F, BLK)


def _make_init(h):
    ept, nb = EPT_H[h], NB_H[h]
    ngrp = nb // NSLOT_G
    rem = nb - ngrp * NSLOT_G

    @functools.partial(
        pl.kernel,
        out_type=jax.ShapeDtypeStruct((HALFS[h], D), jnp.float32),
        mesh=_mesh,
        scratch_types=[
            pltpu.VMEM((nb, BLK), jnp.int32),
            pltpu.VMEM((NSLOT_G, BLK, D), jnp.float32),
            pltpu.VMEM_SHARED((N_NODES, D), jnp.float32),
            pltpu.SemaphoreType.DMA((NSLOT_G,)),
            pltpu.SemaphoreType.DMA((NSLOT_G,)),
            pltpu.SemaphoreType.DMA((NSLOT_G,)),
        ],
    )
    def init_k(e_hbm, v_hbm, srcp, h0, idx_v, bufs, v_sh, sa, sb, sc):
        c = lax.axis_index("c")
        s = lax.axis_index("s")
        base = (c * NS + s) * ept
        ebase = HBASE[h] + base
        pltpu.sync_copy(v_hbm.at[pl.ds(s * 624, 624)],
                        v_sh.at[pl.ds(s * 624, 624)])

        @pl.when(s == NS - 1)
        def _():
            pltpu.sync_copy(v_hbm.at[pl.ds(9984, 16)],
                            v_sh.at[pl.ds(9984, 16)])

        pltpu.sync_copy(srcp.at[c, s], idx_v)
        plsc.subcore_barrier()

        def a_issue(j, p):
            pltpu.async_copy(e_hbm.at[pl.ds(ebase + j * BLK, BLK), :],
                             bufs.at[p], sa.at[p])

        def a_wait(p):
            pltpu.make_async_copy(e_hbm.at[pl.ds(ebase, BLK), :],
                                  bufs.at[p], sa.at[p]).wait()

        def b_issue(j, p):
            pltpu.async_copy(v_sh.at[idx_v.at[j]], bufs.at[p], sb.at[p],
                             add=True)

        def b_wait(p):
            pltpu.make_async_copy(v_sh.at[idx_v.at[0]], bufs.at[p],
                                  sb.at[p]).wait()

        def c_issue(j, p):
            pltpu.async_copy(bufs.at[p],
                             h0.at[pl.ds(base + j * BLK, BLK), :],
                             sc.at[p])

        def c_wait(p):
            pltpu.make_async_copy(bufs.at[p], h0.at[pl.ds(base, BLK), :],
                                  sc.at[p]).wait()

        for p in range(NSLOT_G):
            a_issue(p, p)

        def body(k, _):
            for p in range(NSLOT_G):
                j = k * NSLOT_G + p
                a_wait(p)
                b_issue(j, p)
            for p in range(NSLOT_G):
                j = k * NSLOT_G + p
                b_wait(p)
                c_issue(j, p)
                c_wait(p)
                a_issue(j + NSLOT_G, p)
            return _

        lax.fori_loop(0, ngrp - 1, body, None)
        e0 = (ngrp - 1) * NSLOT_G
        for p in range(NSLOT_G):
            a_wait(p)
            b_issue(e0 + p, p)
        for p in range(NSLOT_G):
            b_wait(p)
            c_issue(e0 + p, p)
            c_wait(p)
            if p < rem:
                a_issue(e0 + NSLOT_G + p, p)
        for p in range(rem):
            a_wait(p)
            b_issue(e0 + NSLOT_G + p, p)
        for p in range(rem):
            b_wait(p)
            c_issue(e0 + NSLOT_G + p, p)
            c_wait(p)

    return init_k


NSLOT_G = 3
MVR = N_NODES // NS


def _make_gather(h):
    ept, nb = EPT_H[h], NB_H[h]
    ngrp = nb // NSLOT_G
    rem = nb - ngrp * NSLOT_G

    @functools.partial(
        pl.kernel,
        out_type=jax.ShapeDtypeStruct((HALFS[h], D), jnp.float32),
        mesh=_mesh,
        scratch_types=[
            pltpu.VMEM((nb, BLK), jnp.int32),
            pltpu.VMEM((nb, BLK), jnp.int32),
            pltpu.VMEM((NSLOT_G, BLK, D), jnp.float32),
            pltpu.VMEM_SHARED((N_NODES, D), jnp.float32),
            pltpu.SemaphoreType.DMA((NSLOT_G,)),
            pltpu.SemaphoreType.DMA((NSLOT_G,)),
            pltpu.SemaphoreType.DMA((NSLOT_G,)),
        ],
    )
    def gather_k(mv, hneg, srcp, revp, out, src_v, rev_v, bufs, mv_sh,
                 sa, sb, sc):
        c = lax.axis_index("c")
        s = lax.axis_index("s")
        base = (c * NS + s) * ept
        pltpu.sync_copy(mv.at[pl.ds(s * 624, 624)],
                        mv_sh.at[pl.ds(s * 624, 624)])

        @pl.when(s == NS - 1)
        def _():
            pltpu.sync_copy(mv.at[pl.ds(9984, 16)],
                            mv_sh.at[pl.ds(9984, 16)])

        pltpu.sync_copy(srcp.at[c, s], src_v)
        pltpu.sync_copy(revp.at[c, s], rev_v)
        plsc.subcore_barrier()

        def a_issue(j, p):
            pltpu.async_copy(mv_sh.at[src_v.at[j]], bufs.at[p], sa.at[p])

        def a_wait(p):
            pltpu.make_async_copy(mv_sh.at[src_v.at[0]], bufs.at[p],
                                  sa.at[p]).wait()

        def b_issue(j, p):
            pltpu.async_copy(hneg.at[rev_v.at[j]], bufs.at[p], sb.at[p],
                             add=True)

        def b_wait(p):
            pltpu.make_async_copy(hneg.at[rev_v.at[0]], bufs.at[p],
                                  sb.at[p]).wait()

        def c_issue(j, p):
            pltpu.async_copy(bufs.at[p],
                             out.at[pl.ds(base + j * BLK, BLK), :],
                             sc.at[p])

        def c_wait(p):
            pltpu.make_async_copy(bufs.at[p], out.at[pl.ds(base, BLK), :],
                                  sc.at[p]).wait()

        for p in range(NSLOT_G):
            a_issue(p, p)

        def body(k, _):
            for p in range(NSLOT_G):
                j = k * NSLOT_G + p
                a_wait(p)
                b_issue(j, p)
            for p in range(NSLOT_G):
                j = k * NSLOT_G + p
                b_wait(p)
                c_issue(j, p)
                c_wait(p)
                a_issue(j + NSLOT_G, p)
            return _

        lax.fori_loop(0, ngrp - 1, body, None)
        e0 = (ngrp - 1) * NSLOT_G
        for p in range(NSLOT_G):
            a_wait(p)
            b_issue(e0 + p, p)
        for p in range(NSLOT_G):
            b_wait(p)
            c_issue(e0 + p, p)
            c_wait(p)
            if p < rem:
                a_issue(e0 + NSLOT_G + p, p)
        for p in range(rem):
            a_wait(p)
            b_issue(e0 + NSLOT_G + p, p)
        for p in range(rem):
            b_wait(p)
            c_issue(e0 + NSLOT_G + p, p)
            c_wait(p)

    return gather_k


_sc_init_h = [_make_init(0), _make_init(1)]
_sc_gather_h = [_make_gather(0), _make_gather(1)]


@functools.partial(
    pl.kernel,
    out_type=jax.ShapeDtypeStruct((NC, NPAD, D), jnp.float32),
    mesh=_mesh,
    scratch_types=[
        pltpu.VMEM((NB_F, BLK), jnp.int32),
        pltpu.VMEM((NSLOT_S, BLK, D), jnp.float32),
        pltpu.VMEM_SHARED((NPAD, D), jnp.float32),
        pltpu.SemaphoreType.DMA((NSLOT_S,)),
        pltpu.SemaphoreType.DMA((NSLOT_S,)),
    ],
)
def _sc_scatter(data, idxp, zeros, out, idx_v, bufs, acc_sh, sa, sb):
    c = lax.axis_index("c")
    s = lax.axis_index("s")
    base = (c * NS + s) * EPT_F
    pltpu.sync_copy(zeros.at[pl.ds(s * STRIPE, STRIPE)],
                    acc_sh.at[pl.ds(s * STRIPE, STRIPE)])
    pltpu.sync_copy(idxp.at[c, s], idx_v)
    plsc.subcore_barrier()

    def a_issue(j, p):
        pltpu.async_copy(data.at[pl.ds(base + j * BLK, BLK), :],
                         bufs.at[p], sa.at[p])

    def a_wait(p):
        pltpu.make_async_copy(data.at[pl.ds(base, BLK), :],
                              bufs.at[p], sa.at[p]).wait()

    def b_issue(j, p):
        pltpu.async_copy(bufs.at[p], acc_sh.at[idx_v.at[j]], sb.at[p],
                         add=True)

    def b_wait(p):
        pltpu.make_async_copy(bufs.at[p], acc_sh.at[idx_v.at[0]],
                              sb.at[p]).wait()

    for p in range(NSLOT_S):
        a_issue(p, p)

    def body(k, _):
        for p in range(NSLOT_S):
            j = k * NSLOT_S + p
            a_wait(p)
            b_issue(j, p)
        for p in range(NSLOT_S):
            b_wait(p)
            a_issue(k * NSLOT_S + p + NSLOT_S, p)
        return _

    lax.fori_loop(0, NGRP_S, body, None)
    e0 = NGRP_S * NSLOT_S
    for p in range(NSLOT_S):
        a_wait(p)
        b_issue(e0 + p, p)
    for p in range(NB_F - e0 - NSLOT_S):
        b_wait(p)
        a_issue(e0 + NSLOT_S + p, p)
    b_wait(NSLOT_S - 1)
    for p in range(NB_F - e0 - NSLOT_S):
        a_wait(p)
        b_issue(e0 + NSLOT_S + p, p)
    for p in range(NB_F - e0 - NSLOT_S):
        b_wait(p)
    plsc.subcore_barrier()
    pltpu.sync_copy(acc_sh.at[pl.ds(s * STRIPE, STRIPE)],
                    out.at[c, pl.ds(s * STRIPE, STRIPE)])


BE = 1600
HGRIDS = tuple((n // BE,) for n in HALFS)
OFF1 = HALF1 // BE


def _tc_combine_body(p_ref, o_ref):
    o_ref[...] = -(p_ref[0] + p_ref[1])


def _tc_out_body(p_ref, o_ref):
    o_ref[...] = p_ref[0] + p_ref[1]


_half_spec = pl.BlockSpec((BE, D), lambda i: (i, 0))
_w_spec = pl.BlockSpec((D, D), lambda i: (0, 0))
_b_spec = pl.BlockSpec((1, D), lambda i: (0, 0))
_alias_spec = pl.BlockSpec((8, D), lambda i: (0, 0))


def _make_relu_neg(h):
    if h == 0:
        def body0(x_ref, o_ref):
            o_ref[...] = -jnp.maximum(x_ref[...], 0.0)

        return pl.pallas_call(
            body0,
            grid=HGRIDS[0],
            in_specs=[_half_spec],
            out_specs=pl.BlockSpec((BE, D), lambda i: (i, 0)),
            out_shape=jax.ShapeDtypeStruct((N_EDGES, D), jnp.float32),
        )

    def body1(x_ref, prev_ref, o_ref):
        del prev_ref
        o_ref[...] = -jnp.maximum(x_ref[...], 0.0)

    return pl.pallas_call(
        body1,
        grid=HGRIDS[1],
        in_specs=[_half_spec, _alias_spec],
        out_specs=pl.BlockSpec((BE, D), lambda i: (i + OFF1, 0)),
        out_shape=jax.ShapeDtypeStruct((N_EDGES, D), jnp.float32),
        input_output_aliases={1: 0},
    )


def _make_layer(h):
    def compute(m_ref, h_ref, w_ref, b_ref):
        hn = (h_ref[...] + b_ref[...]
              + jnp.dot(m_ref[...], w_ref[...],
                        preferred_element_type=jnp.float32))
        return hn

    if h == 0:
        def body0(m_ref, h_ref, w_ref, b_ref, hn_ref, hneg_ref):
            hn = compute(m_ref, h_ref, w_ref, b_ref)
            hn_ref[...] = hn
            hneg_ref[...] = -jnp.maximum(hn, 0.0)

        return pl.pallas_call(
            body0,
            grid=HGRIDS[0],
            in_specs=[_half_spec, _half_spec, _w_spec, _b_spec],
            out_specs=(_half_spec, pl.BlockSpec((BE, D), lambda i: (i, 0))),
            out_shape=(jax.ShapeDtypeStruct((HALFS[0], D), jnp.float32),
                       jax.ShapeDtypeStruct((N_EDGES, D), jnp.float32)),
        )

    def body1(m_ref, h_ref, w_ref, b_ref, prev_ref, hn_ref, hneg_ref):
        del prev_ref
        hn = compute(m_ref, h_ref, w_ref, b_ref)
        hn_ref[...] = hn
        hneg_ref[...] = -jnp.maximum(hn, 0.0)

    return pl.pallas_call(
        body1,
        grid=HGRIDS[1],
        in_specs=[_half_spec, _half_spec, _w_spec, _b_spec, _alias_spec],
        out_specs=(_half_spec,
                   pl.BlockSpec((BE, D), lambda i: (i + OFF1, 0))),
        out_shape=(jax.ShapeDtypeStruct((HALFS[1], D), jnp.float32),
                   jax.ShapeDtypeStruct((N_EDGES, D), jnp.float32)),
        input_output_aliases={4: 1},
    )


def _make_layer_last(h):
    if h == 0:
        def body0(m_ref, h_ref, w_ref, b_ref, hn_ref):
            hn_ref[...] = (h_ref[...] + b_ref[...]
                           + jnp.dot(m_ref[...], w_ref[...],
                                     preferred_element_type=jnp.float32))

        return pl.pallas_call(
            body0,
            grid=HGRIDS[0],
            in_specs=[_half_spec, _half_spec, _w_spec, _b_spec],
            out_specs=pl.BlockSpec((BE, D), lambda i: (i, 0)),
            out_shape=jax.ShapeDtypeStruct((N_EDGES, D), jnp.float32),
        )

    def body1(m_ref, h_ref, w_ref, b_ref, prev_ref, hn_ref):
        del prev_ref
        hn_ref[...] = (h_ref[...] + b_ref[...]
                       + jnp.dot(m_ref[...], w_ref[...],
                                 preferred_element_type=jnp.float32))

    return pl.pallas_call(
        body1,
        grid=HGRIDS[1],
        in_specs=[_half_spec, _half_spec, _w_spec, _b_spec, _alias_spec],
        out_specs=pl.BlockSpec((BE, D), lambda i: (i + OFF1, 0)),
        out_shape=jax.ShapeDtypeStruct((N_EDGES, D), jnp.float32),
        input_output_aliases={4: 0},
    )


_tc_relu_neg_h = [_make_relu_neg(0), _make_relu_neg(1)]
_tc_layer_h = [_make_layer(0), _make_layer(1)]
_tc_layer_last_h = [_make_layer_last(0), _make_layer_last(1)]

_tc_combine = pl.pallas_call(
    _tc_combine_body,
    grid=(8,),
    in_specs=[pl.BlockSpec((NC, NPAD // 8, D), lambda i: (0, i, 0))],
    out_specs=pl.BlockSpec((NPAD // 8, D), lambda i: (i, 0)),
    out_shape=jax.ShapeDtypeStruct((NPAD, D), jnp.float32),
)

_tc_out = pl.pallas_call(
    _tc_out_body,
    grid=(10,),
    in_specs=[pl.BlockSpec((NC, N_NODES // 10, D), lambda i: (0, i, 0))],
    out_specs=pl.BlockSpec((N_NODES // 10, D), lambda i: (i, 0)),
    out_shape=jax.ShapeDtypeStruct((N_NODES, D), jnp.float32),
)


def kernel(V, E, edge_index, rev_index, W1, b1, W2, b2, W3, b3):
    src = edge_index[0]
    dest = edge_index[1]
    srcp_h = [_half_idx(src, h) for h in range(2)]
    revp_h = [_half_idx(rev_index, h) for h in range(2)]
    destp = _full_idx(dest)
    srcp_f = _full_idx(src)
    zeros = jnp.zeros((NPAD, D), jnp.float32)

    h_half = [_sc_init_h[h](E, V, srcp_h[h]) for h in range(2)]
    hneg_v = _tc_relu_neg_h[0](h_half[0])
    hneg = _tc_relu_neg_h[1](h_half[1], hneg_v)

    params = [(W1, b1.reshape(1, D)), (W2, b2.reshape(1, D)),
              (W3, b3.reshape(1, D))]
    for li, (w, b2d) in enumerate(params):
        parts = _sc_scatter(hneg, destp, zeros)
        mv = _tc_combine(parts)
        m_half = [_sc_gather_h[h](mv, hneg, srcp_h[h], revp_h[h])
                  for h in range(2)]
        if li < 2:
            h0n, hneg_v = _tc_layer_h[0](m_half[0], h_half[0], w, b2d)
            h1n, hneg = _tc_layer_h[1](m_half[1], h_half[1], w, b2d, hneg_v)
            h_half = [h0n, h1n]
        else:
            h_v = _tc_layer_last_h[0](m_half[0], h_half[0], w, b2d)
            h_full = _tc_layer_last_h[1](m_half[1], h_half[1], w, b2d, h_v)
    parts = _sc_scatter(h_full, srcp_f, zeros)
    v_out = _tc_out(parts)
    return (v_out, h_full)
